# baseline (speedup 1.0000x reference)
import functools

import jax
import jax.numpy as jnp
from jax import lax
from jax.experimental import pallas as pl
from jax.experimental.pallas import tpu as pltpu
from jax.experimental.pallas import tpu_sc as plsc

N = 100000
D = 128
OUT = 128
K = 4

NC = 2
NS = 16
NW = NC * NS

RPW = 3136
LAST = N - (NW - 1) * RPW
C = 32
G_FULL = RPW // C
G_LAST = LAST // C
assert RPW % C == 0 and LAST % C == 0

BT = 2000
assert N % BT == 0


def _rne_bf16_bits(x):
    u = jax.lax.bitcast_convert_type(x, jnp.uint32)
    r = ((u >> 16) & jnp.uint32(1)) + jnp.uint32(0x7FFF)
    return u + r


def _tc_body(feat_ref, prev_ref, w0t_ref, wba_ref, wbb_ref, b_ref, a_ref, p_ref):
    feat = feat_ref[...].astype(jnp.bfloat16)
    prev = prev_ref[...].astype(jnp.bfloat16)
    a_ref[...] = (
        jnp.dot(feat, w0t_ref[...], preferred_element_type=jnp.float32)
        + b_ref[0][None, :]
    )

    xa = jnp.dot(prev, wba_ref[...], preferred_element_type=jnp.float32)
    xb = jnp.dot(prev, wbb_ref[...], preferred_element_type=jnp.float32)
    lo = _rne_bf16_bits(xa) >> 16
    hi = _rne_bf16_bits(xb) & jnp.uint32(0xFFFF0000)
    word = jax.lax.bitcast_convert_type(hi | lo, jnp.int32)
    p_ref[...] = word.reshape(2 * BT, OUT)


def _tc_phase(features, prev_features, w0t, wba, wbb, b8):
    grid = (N // BT,)
    return pl.pallas_call(
        _tc_body,
        grid=grid,
        in_specs=[
            pl.BlockSpec((BT, D), lambda i: (i, 0)),
            pl.BlockSpec((BT, D), lambda i: (i, 0)),
            pl.BlockSpec((D, OUT), lambda i: (0, 0)),
            pl.BlockSpec((D, K * OUT // 2), lambda i: (0, 0)),
            pl.BlockSpec((D, K * OUT // 2), lambda i: (0, 0)),
            pl.BlockSpec((16, OUT), lambda i: (0, 0)),
        ],
        out_specs=[
            pl.BlockSpec((BT, OUT), lambda i: (i, 0)),
            pl.BlockSpec((2 * BT, OUT), lambda i: (i, 0)),
        ],
        out_shape=[
            jax.ShapeDtypeStruct((N, OUT), jnp.float32),
            jax.ShapeDtypeStruct((2 * N, OUT), jnp.int32),
        ],
    )(features, prev_features, w0t, wba, wbb, b8)


def _sc_gather_sum(p_flat, a_full, idx_flat):
    mesh = plsc.VectorSubcoreMesh(core_axis_name="c", subcore_axis_name="s")

    @functools.partial(
        pl.kernel,
        out_type=jax.ShapeDtypeStruct((N, OUT), jnp.float32),
        mesh=mesh,
        scratch_types=[
            pltpu.VMEM((RPW * K,), jnp.int32),
            pltpu.VMEM((K * C, OUT // 2), jnp.int32),
            pltpu.VMEM((K * C, OUT // 2), jnp.int32),
            pltpu.VMEM((C, OUT), jnp.float32),
            pltpu.VMEM((C, OUT), jnp.float32),
            pltpu.VMEM((C, OUT), jnp.float32),
            pltpu.VMEM((C, OUT), jnp.float32),
            pltpu.SemaphoreType.DMA,
            pltpu.SemaphoreType.DMA,
            pltpu.SemaphoreType.DMA,
            pltpu.SemaphoreType.DMA,
            pltpu.SemaphoreType.DMA,
            pltpu.SemaphoreType.DMA,
        ],
        compiler_params=pltpu.CompilerParams(use_tc_tiling_on_sc=False),
    )
    def sc_kernel(p_hbm, a_hbm, idx_hbm, out_hbm,
                  idxall, gb0, gb1, av0, av1, ov0, ov1,
                  sg0, sg1, sa0, sa1, so0, so1):
        cid = lax.axis_index("c")
        sid = lax.axis_index("s")
        wid = sid * NC + cid
        base = wid * RPW
        num_g = jnp.where(wid == NW - 1, G_LAST, G_FULL)
        kpat = lax.rem(lax.iota(jnp.int32, 16), 4)

        pltpu.sync_copy(
            idx_hbm.at[pl.ds(base * K, LAST * K)], idxall.at[pl.ds(0, LAST * K)])

        @pl.when(wid < NW - 1)
        def _():
            pltpu.sync_copy(
                idx_hbm.at[pl.ds(base * K + LAST * K, (RPW - LAST) * K)],
                idxall.at[pl.ds(LAST * K, (RPW - LAST) * K)])

        def conv(v, carry):
            sl = pl.ds(v * 16, 16)
            idxall[sl] = idxall[sl] * 4 + kpat
            return carry

        lax.fori_loop(0, RPW * K // 16, conv, 0)

        def start(g, gb, av, sg, sa):
            pltpu.async_copy(
                p_hbm.at[idxall.at[pl.ds(g * K * C, K * C)]], gb, sg)
            pltpu.async_copy(a_hbm.at[pl.ds(base + g * C, C)], av, sa)

        def finish(g, gb, av, ov, sg, sa, so):
            pltpu.make_async_copy(
                p_hbm.at[idxall.at[pl.ds(g * K * C, K * C)]], gb, sg).wait()
            pltpu.make_async_copy(
                a_hbm.at[pl.ds(base + g * C, C)], av, sa).wait()

            @pl.when(g >= 2)
            def _():
                pltpu.make_async_copy(
                    ov, out_hbm.at[pl.ds(base, C)], so).wait()

            hi_mask = jnp.int32(-65536)
            sixteen = jnp.full((16,), 16, dtype=jnp.int32)

            def row(c, carry):
                for t in range(OUT // 32):
                    slo = pl.ds(16 * t, 16)
                    shi = pl.ds(OUT // 2 + 16 * t, 16)
                    acc_lo = av[c, slo]
                    acc_hi = av[c, shi]
                    for k in range(K):
                        w = gb[4 * c + k, pl.ds(16 * t, 16)]
                        acc_lo = acc_lo + lax.bitcast_convert_type(
                            lax.shift_left(w, sixteen), jnp.float32)
                        acc_hi = acc_hi + lax.bitcast_convert_type(
                            lax.bitwise_and(w, hi_mask), jnp.float32)
                    ov[c, slo] = acc_lo
                    ov[c, shi] = acc_hi
                return carry

            lax.fori_loop(0, C, row, 0)
            pltpu.async_copy(ov, out_hbm.at[pl.ds(base + g * C, C)], so)

        start(0, gb0, av0, sg0, sa0)

        def pair(p, carry):
            g0 = p * 2
            start(g0 + 1, gb1, av1, sg1, sa1)
            finish(g0, gb0, av0, ov0, sg0, sa0, so0)

            @pl.when(g0 + 2 < num_g)
            def _():
                start(g0 + 2, gb0, av0, sg0, sa0)

            finish(g0 + 1, gb1, av1, ov1, sg1, sa1, so1)
            return carry

        lax.fori_loop(0, num_g // 2, pair, 0)

        @pl.when(num_g % 2 == 1)
        def _():
            finish(num_g - 1, gb0, av0, ov0, sg0, sa0, so0)

        pltpu.make_async_copy(ov0, out_hbm.at[pl.ds(base, C)], so0).wait()
        pltpu.make_async_copy(ov1, out_hbm.at[pl.ds(base, C)], so1).wait()

    return sc_kernel(p_flat, a_full, idx_flat)


def kernel(features, prev_features, neighbor_idx, W, b):
    idx_flat = neighbor_idx.astype(jnp.int32).reshape(N * K)

    w0t = W[:, :D].T
    wb_base = W[:, D:].reshape(OUT, K, D).transpose(2, 1, 0)
    wba = wb_base[:, :, : OUT // 2].reshape(D, K * OUT // 2).astype(jnp.bfloat16)
    wbb = wb_base[:, :, OUT // 2:].reshape(D, K * OUT // 2).astype(jnp.bfloat16)
    w0t = w0t.astype(jnp.bfloat16)
    b8 = jnp.broadcast_to(b[None, :], (16, OUT))

    a_full, p_blk = _tc_phase(features, prev_features, w0t, wba, wbb, b8)
    p_flat = p_blk.reshape(N * K, OUT // 2)

    return _sc_gather_sum(p_flat, a_full, idx_flat)


# speedup vs baseline: 1.0066x; 1.0066x over previous
import functools

import jax
import jax.numpy as jnp
from jax import lax
from jax.experimental import pallas as pl
from jax.experimental.pallas import tpu as pltpu
from jax.experimental.pallas import tpu_sc as plsc

N = 100000
D = 128
OUT = 128
K = 4

NC = 2
NS = 16
NW = NC * NS

RPW = 3136
LAST = N - (NW - 1) * RPW
C = 32
G_FULL = RPW // C
G_LAST = LAST // C
assert RPW % C == 0 and LAST % C == 0

BT = 2000
assert N % BT == 0


def _rne_bf16_bits(x):
    u = jax.lax.bitcast_convert_type(x, jnp.uint32)
    r = ((u >> 16) & jnp.uint32(1)) + jnp.uint32(0x7FFF)
    return u + r


def _tc_body(feat_ref, prev_ref, w0t_ref, wba_ref, wbb_ref, b_ref, a_ref, p_ref):
    a_ref[...] = (
        jnp.dot(feat_ref[...], w0t_ref[...], preferred_element_type=jnp.float32)
        + b_ref[0][None, :]
    )
    xa = jnp.dot(prev_ref[...], wba_ref[...], preferred_element_type=jnp.float32)
    xb = jnp.dot(prev_ref[...], wbb_ref[...], preferred_element_type=jnp.float32)
    lo = _rne_bf16_bits(xa) >> 16
    hi = _rne_bf16_bits(xb) & jnp.uint32(0xFFFF0000)
    word = jax.lax.bitcast_convert_type(hi | lo, jnp.int32)
    p_ref[...] = word.reshape(2 * BT, OUT)


def _tc_phase(features, prev_features, w0t, wba, wbb, b8):
    grid = (N // BT,)
    return pl.pallas_call(
        _tc_body,
        grid=grid,
        in_specs=[
            pl.BlockSpec((BT, D), lambda i: (i, 0)),
            pl.BlockSpec((BT, D), lambda i: (i, 0)),
            pl.BlockSpec((D, OUT), lambda i: (0, 0)),
            pl.BlockSpec((D, K * OUT // 2), lambda i: (0, 0)),
            pl.BlockSpec((D, K * OUT // 2), lambda i: (0, 0)),
            pl.BlockSpec((8, OUT), lambda i: (0, 0)),
        ],
        out_specs=[
            pl.BlockSpec((BT, OUT), lambda i: (i, 0)),
            pl.BlockSpec((2 * BT, OUT), lambda i: (i, 0)),
        ],
        out_shape=[
            jax.ShapeDtypeStruct((N, OUT), jnp.float32),
            jax.ShapeDtypeStruct((2 * N, OUT), jnp.int32),
        ],
    )(features, prev_features, w0t, wba, wbb, b8)


def _sc_gather_sum(p_flat, a_full, idx_flat):
    mesh = plsc.VectorSubcoreMesh(core_axis_name="c", subcore_axis_name="s")

    @functools.partial(
        pl.kernel,
        out_type=jax.ShapeDtypeStruct((N, OUT), jnp.float32),
        mesh=mesh,
        scratch_types=[
            pltpu.VMEM((RPW * K,), jnp.int32),
            pltpu.VMEM((K * C, OUT // 2), jnp.int32),
            pltpu.VMEM((K * C, OUT // 2), jnp.int32),
            pltpu.VMEM((C, OUT), jnp.float32),
            pltpu.VMEM((C, OUT), jnp.float32),
            pltpu.VMEM((C, OUT), jnp.float32),
            pltpu.VMEM((C, OUT), jnp.float32),
            pltpu.SemaphoreType.DMA,
            pltpu.SemaphoreType.DMA,
            pltpu.SemaphoreType.DMA,
            pltpu.SemaphoreType.DMA,
            pltpu.SemaphoreType.DMA,
            pltpu.SemaphoreType.DMA,
        ],
        compiler_params=pltpu.CompilerParams(use_tc_tiling_on_sc=False),
    )
    def sc_kernel(p_hbm, a_hbm, idx_hbm, out_hbm,
                  idxall, gb0, gb1, av0, av1, ov0, ov1,
                  sg0, sg1, sa0, sa1, so0, so1):
        cid = lax.axis_index("c")
        sid = lax.axis_index("s")
        wid = sid * NC + cid
        base = wid * RPW
        num_g = jnp.where(wid == NW - 1, G_LAST, G_FULL)
        kpat = lax.rem(lax.iota(jnp.int32, 16), 4)

        pltpu.sync_copy(
            idx_hbm.at[pl.ds(base * K, LAST * K)], idxall.at[pl.ds(0, LAST * K)])

        @pl.when(wid < NW - 1)
        def _():
            pltpu.sync_copy(
                idx_hbm.at[pl.ds(base * K + LAST * K, (RPW - LAST) * K)],
                idxall.at[pl.ds(LAST * K, (RPW - LAST) * K)])

        def conv(v, carry):
            sl = pl.ds(v * 16, 16)
            idxall[sl] = idxall[sl] * 4 + kpat
            return carry

        lax.fori_loop(0, RPW * K // 16, conv, 0)

        def start(g, gb, av, sg, sa):
            pltpu.async_copy(
                p_hbm.at[idxall.at[pl.ds(g * K * C, K * C)]], gb, sg)
            pltpu.async_copy(a_hbm.at[pl.ds(base + g * C, C)], av, sa)

        def finish(g, gb, av, ov, sg, sa, so):
            pltpu.make_async_copy(
                p_hbm.at[idxall.at[pl.ds(g * K * C, K * C)]], gb, sg).wait()
            pltpu.make_async_copy(
                a_hbm.at[pl.ds(base + g * C, C)], av, sa).wait()

            @pl.when(g >= 2)
            def _():
                pltpu.make_async_copy(
                    ov, out_hbm.at[pl.ds(base, C)], so).wait()

            hi_mask = jnp.int32(-65536)
            sixteen = jnp.full((16,), 16, dtype=jnp.int32)

            def row(c, carry):
                for t in range(OUT // 32):
                    slo = pl.ds(16 * t, 16)
                    shi = pl.ds(OUT // 2 + 16 * t, 16)
                    acc_lo = av[c, slo]
                    acc_hi = av[c, shi]
                    for k in range(K):
                        w = gb[4 * c + k, pl.ds(16 * t, 16)]
                        acc_lo = acc_lo + lax.bitcast_convert_type(
                            lax.shift_left(w, sixteen), jnp.float32)
                        acc_hi = acc_hi + lax.bitcast_convert_type(
                            lax.bitwise_and(w, hi_mask), jnp.float32)
                    ov[c, slo] = acc_lo
                    ov[c, shi] = acc_hi
                return carry

            lax.fori_loop(0, C, row, 0)
            pltpu.async_copy(ov, out_hbm.at[pl.ds(base + g * C, C)], so)

        start(0, gb0, av0, sg0, sa0)

        def pair(p, carry):
            g0 = p * 2
            start(g0 + 1, gb1, av1, sg1, sa1)
            finish(g0, gb0, av0, ov0, sg0, sa0, so0)

            @pl.when(g0 + 2 < num_g)
            def _():
                start(g0 + 2, gb0, av0, sg0, sa0)

            finish(g0 + 1, gb1, av1, ov1, sg1, sa1, so1)
            return carry

        lax.fori_loop(0, num_g // 2, pair, 0)

        @pl.when(num_g % 2 == 1)
        def _():
            finish(num_g - 1, gb0, av0, ov0, sg0, sa0, so0)

        pltpu.make_async_copy(ov0, out_hbm.at[pl.ds(base, C)], so0).wait()
        pltpu.make_async_copy(ov1, out_hbm.at[pl.ds(base, C)], so1).wait()

    return sc_kernel(p_flat, a_full, idx_flat)


def kernel(features, prev_features, neighbor_idx, W, b):
    idx_flat = neighbor_idx.astype(jnp.int32).reshape(N * K)

    w0t = W[:, :D].T
    wb_base = W[:, D:].reshape(OUT, K, D).transpose(2, 1, 0)
    wba = wb_base[:, :, : OUT // 2].reshape(D, K * OUT // 2)
    wbb = wb_base[:, :, OUT // 2:].reshape(D, K * OUT // 2)
    b8 = jnp.broadcast_to(b[None, :], (8, OUT))

    a_full, p_blk = _tc_phase(features, prev_features, w0t, wba, wbb, b8)
    p_flat = p_blk.reshape(N * K, OUT // 2)

    return _sc_gather_sum(p_flat, a_full, idx_flat)


# speedup vs baseline: 1.2522x; 1.2440x over previous
import functools

import jax
import jax.numpy as jnp
from jax import lax
from jax.experimental import pallas as pl
from jax.experimental.pallas import tpu as pltpu
from jax.experimental.pallas import tpu_sc as plsc

N = 100000
D = 128
OUT = 128
K = 4

NC = 2
NS = 16
NW = NC * NS

RPW = 3136
LAST = N - (NW - 1) * RPW
C = 32
G_FULL = RPW // C
G_LAST = LAST // C
assert RPW % C == 0 and LAST % C == 0

BT = 2000
assert N % BT == 0


def _rne_bf16_bits(x):
    u = jax.lax.bitcast_convert_type(x, jnp.uint32)
    r = ((u >> 16) & jnp.uint32(1)) + jnp.uint32(0x7FFF)
    return u + r


def _tc_body(feat_ref, prev_ref, w0t_ref, wba_ref, wbb_ref, b_ref, a_ref, p_ref):
    a_ref[...] = (
        jnp.dot(feat_ref[...], w0t_ref[...], preferred_element_type=jnp.float32)
        + b_ref[0][None, :]
    )
    xa = jnp.dot(prev_ref[...], wba_ref[...], preferred_element_type=jnp.float32)
    xb = jnp.dot(prev_ref[...], wbb_ref[...], preferred_element_type=jnp.float32)
    lo = _rne_bf16_bits(xa) >> 16
    hi = _rne_bf16_bits(xb) & jnp.uint32(0xFFFF0000)
    word = jax.lax.bitcast_convert_type(hi | lo, jnp.int32)
    p_ref[...] = word.reshape(2 * BT, OUT)


def _tc_phase(features, prev_features, w0t, wba, wbb, b8):
    grid = (N // BT,)
    return pl.pallas_call(
        _tc_body,
        grid=grid,
        in_specs=[
            pl.BlockSpec((BT, D), lambda i: (i, 0)),
            pl.BlockSpec((BT, D), lambda i: (i, 0)),
            pl.BlockSpec((D, OUT), lambda i: (0, 0)),
            pl.BlockSpec((D, K * OUT // 2), lambda i: (0, 0)),
            pl.BlockSpec((D, K * OUT // 2), lambda i: (0, 0)),
            pl.BlockSpec((8, OUT), lambda i: (0, 0)),
        ],
        out_specs=[
            pl.BlockSpec((BT, OUT), lambda i: (i, 0)),
            pl.BlockSpec((2 * BT, OUT), lambda i: (i, 0)),
        ],
        out_shape=[
            jax.ShapeDtypeStruct((N, OUT), jnp.float32),
            jax.ShapeDtypeStruct((2 * N, OUT), jnp.int32),
        ],
    )(features, prev_features, w0t, wba, wbb, b8)


def _sc_gather_sum(p_flat, a_full, idx_flat):
    mesh = plsc.VectorSubcoreMesh(core_axis_name="c", subcore_axis_name="s")

    @functools.partial(
        pl.kernel,
        out_type=jax.ShapeDtypeStruct((N, OUT), jnp.float32),
        mesh=mesh,
        scratch_types=[
            pltpu.VMEM((K * RPW,), jnp.int32),
            pltpu.VMEM((K, C, OUT // 2), jnp.int32),
            pltpu.VMEM((K, C, OUT // 2), jnp.int32),
            pltpu.VMEM((C, OUT), jnp.float32),
            pltpu.VMEM((C, OUT), jnp.float32),
            pltpu.VMEM((C, OUT), jnp.float32),
            pltpu.VMEM((C, OUT), jnp.float32),
            pltpu.SemaphoreType.DMA,
            pltpu.SemaphoreType.DMA,
            pltpu.SemaphoreType.DMA,
            pltpu.SemaphoreType.DMA,
            pltpu.SemaphoreType.DMA,
            pltpu.SemaphoreType.DMA,
        ],
        compiler_params=pltpu.CompilerParams(use_tc_tiling_on_sc=False),
    )
    def sc_kernel(p_hbm, a_hbm, idx_hbm, out_hbm,
                  idxall, gb0, gb1, av0, av1, ov0, ov1,
                  sg0, sg1, sa0, sa1, so0, so1):
        cid = lax.axis_index("c")
        sid = lax.axis_index("s")
        wid = sid * NC + cid
        base = wid * RPW
        num_g = jnp.where(wid == NW - 1, G_LAST, G_FULL)

        for k in range(K):
            pltpu.sync_copy(
                idx_hbm.at[pl.ds(k * N + base, LAST)],
                idxall.at[pl.ds(k * RPW, LAST)])

        @pl.when(wid < NW - 1)
        def _():
            for k in range(K):
                pltpu.sync_copy(
                    idx_hbm.at[pl.ds(k * N + base + LAST, RPW - LAST)],
                    idxall.at[pl.ds(k * RPW + LAST, RPW - LAST)])

        def conv(v, carry):
            for k in range(K):
                sl = pl.ds(k * RPW + v * 16, 16)
                idxall[sl] = idxall[sl] * 4 + jnp.int32(k)
            return carry

        lax.fori_loop(0, RPW // 16, conv, 0)

        def start(g, gb, av, sg, sa):
            for k in range(K):
                pltpu.async_copy(
                    p_hbm.at[idxall.at[pl.ds(k * RPW + g * C, C)]],
                    gb.at[k], sg)
            pltpu.async_copy(a_hbm.at[pl.ds(base + g * C, C)], av, sa)

        def finish(g, gb, av, ov, sg, sa, so):
            for k in range(K):
                pltpu.make_async_copy(
                    p_hbm.at[idxall.at[pl.ds(k * RPW + g * C, C)]],
                    gb.at[k], sg).wait()
            pltpu.make_async_copy(
                a_hbm.at[pl.ds(base + g * C, C)], av, sa).wait()

            @pl.when(g >= 2)
            def _():
                pltpu.make_async_copy(
                    ov, out_hbm.at[pl.ds(base, C)], so).wait()

            hi_mask = jnp.int32(-65536)
            sixteen = jnp.full((16,), 16, dtype=jnp.int32)

            def row(c, carry):
                for t in range(OUT // 32):
                    slo = pl.ds(16 * t, 16)
                    shi = pl.ds(OUT // 2 + 16 * t, 16)
                    acc_lo = av[c, slo]
                    acc_hi = av[c, shi]
                    for k in range(K):
                        w = gb[k, c, pl.ds(16 * t, 16)]
                        acc_lo = acc_lo + lax.bitcast_convert_type(
                            lax.shift_left(w, sixteen), jnp.float32)
                        acc_hi = acc_hi + lax.bitcast_convert_type(
                            lax.bitwise_and(w, hi_mask), jnp.float32)
                    ov[c, slo] = acc_lo
                    ov[c, shi] = acc_hi
                return carry

            lax.fori_loop(0, C, row, 0)
            pltpu.async_copy(ov, out_hbm.at[pl.ds(base + g * C, C)], so)

        start(0, gb0, av0, sg0, sa0)

        def pair(p, carry):
            g0 = p * 2
            start(g0 + 1, gb1, av1, sg1, sa1)
            finish(g0, gb0, av0, ov0, sg0, sa0, so0)

            @pl.when(g0 + 2 < num_g)
            def _():
                start(g0 + 2, gb0, av0, sg0, sa0)

            finish(g0 + 1, gb1, av1, ov1, sg1, sa1, so1)
            return carry

        lax.fori_loop(0, num_g // 2, pair, 0)

        @pl.when(num_g % 2 == 1)
        def _():
            finish(num_g - 1, gb0, av0, ov0, sg0, sa0, so0)

        pltpu.make_async_copy(ov0, out_hbm.at[pl.ds(base, C)], so0).wait()
        pltpu.make_async_copy(ov1, out_hbm.at[pl.ds(base, C)], so1).wait()

    return sc_kernel(p_flat, a_full, idx_flat)


def kernel(features, prev_features, neighbor_idx, W, b):
    idx_flat = neighbor_idx.astype(jnp.int32).T.reshape(K * N)

    w0t = W[:, :D].T
    wb_base = W[:, D:].reshape(OUT, K, D).transpose(2, 1, 0)
    wba = wb_base[:, :, : OUT // 2].reshape(D, K * OUT // 2)
    wbb = wb_base[:, :, OUT // 2:].reshape(D, K * OUT // 2)
    b8 = jnp.broadcast_to(b[None, :], (8, OUT))

    a_full, p_blk = _tc_phase(features, prev_features, w0t, wba, wbb, b8)
    p_flat = p_blk.reshape(N * K, OUT // 2)

    return _sc_gather_sum(p_flat, a_full, idx_flat)


# speedup vs baseline: 1.3335x; 1.0649x over previous
import functools

import jax
import jax.numpy as jnp
from jax import lax
from jax.experimental import pallas as pl
from jax.experimental.pallas import tpu as pltpu
from jax.experimental.pallas import tpu_sc as plsc

N = 100000
D = 128
OUT = 128
K = 4

NC = 2
NS = 16
NW = NC * NS

RPW = 3136
LAST = N - (NW - 1) * RPW
C = 32
G_FULL = RPW // C
G_LAST = LAST // C
assert RPW % C == 0 and LAST % C == 0

BT = 4000
assert N % BT == 0


def _rne_bf16_bits(x):
    u = jax.lax.bitcast_convert_type(x, jnp.uint32)
    r = ((u >> 16) & jnp.uint32(1)) + jnp.uint32(0x7FFF)
    return u + r


def _tc_body(feat_ref, prev_ref, w0t_ref, wba_ref, wbb_ref, b_ref, a_ref, p_ref):
    a_ref[...] = (
        jnp.dot(feat_ref[...], w0t_ref[...], preferred_element_type=jnp.float32)
        + b_ref[0][None, :]
    )
    xa = jnp.dot(prev_ref[...], wba_ref[...], preferred_element_type=jnp.float32)
    xb = jnp.dot(prev_ref[...], wbb_ref[...], preferred_element_type=jnp.float32)
    lo = _rne_bf16_bits(xa) >> 16
    hi = _rne_bf16_bits(xb) & jnp.uint32(0xFFFF0000)
    word = jax.lax.bitcast_convert_type(hi | lo, jnp.int32)
    p_ref[...] = word.reshape(2 * BT, OUT)


def _tc_phase(features, prev_features, w0t, wba, wbb, b8):
    grid = (N // BT,)
    return pl.pallas_call(
        _tc_body,
        grid=grid,
        in_specs=[
            pl.BlockSpec((BT, D), lambda i: (i, 0)),
            pl.BlockSpec((BT, D), lambda i: (i, 0)),
            pl.BlockSpec((D, OUT), lambda i: (0, 0)),
            pl.BlockSpec((D, K * OUT // 2), lambda i: (0, 0)),
            pl.BlockSpec((D, K * OUT // 2), lambda i: (0, 0)),
            pl.BlockSpec((8, OUT), lambda i: (0, 0)),
        ],
        out_specs=[
            pl.BlockSpec((BT, OUT), lambda i: (i, 0)),
            pl.BlockSpec((2 * BT, OUT), lambda i: (i, 0)),
        ],
        out_shape=[
            jax.ShapeDtypeStruct((N, OUT), jnp.float32),
            jax.ShapeDtypeStruct((2 * N, OUT), jnp.int32),
        ],
    )(features, prev_features, w0t, wba, wbb, b8)


def _sc_gather_sum(p_flat, a_full, idx_flat):
    mesh = plsc.VectorSubcoreMesh(core_axis_name="c", subcore_axis_name="s")

    @functools.partial(
        pl.kernel,
        out_type=jax.ShapeDtypeStruct((N, OUT), jnp.float32),
        mesh=mesh,
        scratch_types=[
            pltpu.VMEM((K * RPW,), jnp.int32),
            pltpu.VMEM((K, C, OUT // 2), jnp.int32),
            pltpu.VMEM((K, C, OUT // 2), jnp.int32),
            pltpu.VMEM((C, OUT), jnp.float32),
            pltpu.VMEM((C, OUT), jnp.float32),
            pltpu.VMEM((C, OUT), jnp.float32),
            pltpu.VMEM((C, OUT), jnp.float32),
            pltpu.SemaphoreType.DMA,
            pltpu.SemaphoreType.DMA,
            pltpu.SemaphoreType.DMA,
            pltpu.SemaphoreType.DMA,
            pltpu.SemaphoreType.DMA,
            pltpu.SemaphoreType.DMA,
        ],
        compiler_params=pltpu.CompilerParams(use_tc_tiling_on_sc=False),
    )
    def sc_kernel(p_hbm, a_hbm, idx_hbm, out_hbm,
                  idxall, gb0, gb1, av0, av1, ov0, ov1,
                  sg0, sg1, sa0, sa1, so0, so1):
        cid = lax.axis_index("c")
        sid = lax.axis_index("s")
        wid = sid * NC + cid
        base = wid * RPW
        num_g = jnp.where(wid == NW - 1, G_LAST, G_FULL)

        for k in range(K):
            pltpu.sync_copy(
                idx_hbm.at[pl.ds(k * N + base, LAST)],
                idxall.at[pl.ds(k * RPW, LAST)])

        @pl.when(wid < NW - 1)
        def _():
            for k in range(K):
                pltpu.sync_copy(
                    idx_hbm.at[pl.ds(k * N + base + LAST, RPW - LAST)],
                    idxall.at[pl.ds(k * RPW + LAST, RPW - LAST)])

        def conv(v, carry):
            for k in range(K):
                sl = pl.ds(k * RPW + v * 16, 16)
                idxall[sl] = idxall[sl] * 4 + jnp.int32(k)
            return carry

        lax.fori_loop(0, RPW // 16, conv, 0)

        def start(g, gb, av, sg, sa):
            for k in range(K):
                pltpu.async_copy(
                    p_hbm.at[idxall.at[pl.ds(k * RPW + g * C, C)]],
                    gb.at[k], sg)
            pltpu.async_copy(a_hbm.at[pl.ds(base + g * C, C)], av, sa)

        def finish(g, gb, av, ov, sg, sa, so):
            for k in range(K):
                pltpu.make_async_copy(
                    p_hbm.at[idxall.at[pl.ds(k * RPW + g * C, C)]],
                    gb.at[k], sg).wait()
            pltpu.make_async_copy(
                a_hbm.at[pl.ds(base + g * C, C)], av, sa).wait()

            @pl.when(g >= 2)
            def _():
                pltpu.make_async_copy(
                    ov, out_hbm.at[pl.ds(base, C)], so).wait()

            hi_mask = jnp.int32(-65536)
            sixteen = jnp.full((16,), 16, dtype=jnp.int32)

            def row(c, carry):
                for t in range(OUT // 32):
                    slo = pl.ds(16 * t, 16)
                    shi = pl.ds(OUT // 2 + 16 * t, 16)
                    acc_lo = av[c, slo]
                    acc_hi = av[c, shi]
                    for k in range(K):
                        w = gb[k, c, pl.ds(16 * t, 16)]
                        acc_lo = acc_lo + lax.bitcast_convert_type(
                            lax.shift_left(w, sixteen), jnp.float32)
                        acc_hi = acc_hi + lax.bitcast_convert_type(
                            lax.bitwise_and(w, hi_mask), jnp.float32)
                    ov[c, slo] = acc_lo
                    ov[c, shi] = acc_hi
                return carry

            lax.fori_loop(0, C, row, 0)
            pltpu.async_copy(ov, out_hbm.at[pl.ds(base + g * C, C)], so)

        start(0, gb0, av0, sg0, sa0)

        def pair(p, carry):
            g0 = p * 2
            start(g0 + 1, gb1, av1, sg1, sa1)
            finish(g0, gb0, av0, ov0, sg0, sa0, so0)

            @pl.when(g0 + 2 < num_g)
            def _():
                start(g0 + 2, gb0, av0, sg0, sa0)

            finish(g0 + 1, gb1, av1, ov1, sg1, sa1, so1)
            return carry

        lax.fori_loop(0, num_g // 2, pair, 0)

        @pl.when(num_g % 2 == 1)
        def _():
            finish(num_g - 1, gb0, av0, ov0, sg0, sa0, so0)

        pltpu.make_async_copy(ov0, out_hbm.at[pl.ds(base, C)], so0).wait()
        pltpu.make_async_copy(ov1, out_hbm.at[pl.ds(base, C)], so1).wait()

    return sc_kernel(p_flat, a_full, idx_flat)


def kernel(features, prev_features, neighbor_idx, W, b):
    idx_flat = neighbor_idx.astype(jnp.int32).T.reshape(K * N)

    w0t = W[:, :D].T
    wb_base = W[:, D:].reshape(OUT, K, D).transpose(2, 1, 0)
    wba = wb_base[:, :, : OUT // 2].reshape(D, K * OUT // 2)
    wbb = wb_base[:, :, OUT // 2:].reshape(D, K * OUT // 2)
    b8 = jnp.broadcast_to(b[None, :], (8, OUT))

    a_full, p_blk = _tc_phase(features, prev_features, w0t, wba, wbb, b8)
    p_flat = p_blk.reshape(N * K, OUT // 2)

    return _sc_gather_sum(p_flat, a_full, idx_flat)
